# baseline (speedup 1.0000x reference)
import functools

import jax
import jax.numpy as jnp
from jax import lax
from jax.experimental import pallas as pl
from jax.experimental.pallas import tpu as pltpu
from jax.experimental.pallas import tpu_sc as plsc

_NC = 2
_NS = 16
_NW = _NC * _NS
_CHUNK = 128


@functools.lru_cache(maxsize=None)
def _build_gather(B, UD, ID, PW):
    bw = B // _NW
    nchunk = bw // _CHUNK
    mesh = plsc.VectorSubcoreMesh(core_axis_name="c", subcore_axis_name="s")

    @functools.partial(
        pl.kernel,
        out_type=(
            jax.ShapeDtypeStruct((B, UD), jnp.float32),
            jax.ShapeDtypeStruct((B, PW), jnp.float32),
            jax.ShapeDtypeStruct((B, ID), jnp.float32),
            jax.ShapeDtypeStruct((B, PW), jnp.float32),
        ),
        mesh=mesh,
        scratch_types=[
            pltpu.VMEM((bw,), jnp.int32),
            pltpu.VMEM((bw,), jnp.int32),
            pltpu.VMEM((bw,), jnp.int32),
            pltpu.VMEM((bw,), jnp.int32),
            pltpu.VMEM((bw, UD), jnp.float32),
            pltpu.VMEM((bw, PW), jnp.float32),
            pltpu.VMEM((bw, ID), jnp.float32),
            pltpu.VMEM((bw, PW), jnp.float32),
            pltpu.SemaphoreType.DMA,
        ],
        compiler_params=pltpu.CompilerParams(use_tc_tiling_on_sc=False),
    )
    def gather_k(uids, iids, uq, iq, uenc, ienc, uembp, iembp,
                 ue_o, uemb_o, ie_o, iemb_o,
                 uidx, iidx, uqx, iqx, ue_v, uemb_v, ie_v, iemb_v, sem):
        wid = lax.axis_index("s") * _NC + lax.axis_index("c")
        base = wid * bw
        pltpu.sync_copy(uids.at[pl.ds(base, bw)], uidx)
        pltpu.sync_copy(iids.at[pl.ds(base, bw)], iidx)
        pltpu.sync_copy(uq.at[pl.ds(base, bw)], uqx)
        pltpu.sync_copy(iq.at[pl.ds(base, bw)], iqx)
        cps = []
        for c in range(nchunk):
            s = pl.ds(c * _CHUNK, _CHUNK)
            cps.append(pltpu.async_copy(uenc.at[uidx.at[s]], ue_v.at[s], sem))
            cps.append(pltpu.async_copy(uembp.at[uqx.at[s]], uemb_v.at[s], sem))
            cps.append(pltpu.async_copy(ienc.at[iidx.at[s]], ie_v.at[s], sem))
            cps.append(pltpu.async_copy(iembp.at[iqx.at[s]], iemb_v.at[s], sem))
        for cp in cps:
            cp.wait()
        pltpu.sync_copy(ue_v, ue_o.at[pl.ds(base, bw)])
        pltpu.sync_copy(uemb_v, uemb_o.at[pl.ds(base, bw)])
        pltpu.sync_copy(ie_v, ie_o.at[pl.ds(base, bw)])
        pltpu.sync_copy(iemb_v, iemb_o.at[pl.ds(base, bw)])

    return gather_k


def _make_tc_body(ED):
    def tc_body(ue, uembp, ie, iembp, usel, isel, wut, wub, bu, wit, wib, bi,
                users_o, items_o, ratings_o):
        uemb = jnp.where(usel[...] == 0, uembp[:, :ED], uembp[:, ED:])
        iemb = jnp.where(isel[...] == 0, iembp[:, :ED], iembp[:, ED:])
        u = jnp.dot(ue[...], wut[...], preferred_element_type=jnp.float32)
        u = u + jnp.dot(uemb, wub[...], preferred_element_type=jnp.float32)
        u = u + bu[...]
        t = jnp.dot(ie[...], wit[...], preferred_element_type=jnp.float32)
        t = t + jnp.dot(iemb, wib[...], preferred_element_type=jnp.float32)
        t = t + bi[...]
        users_o[...] = u
        items_o[...] = t
        ratings_o[...] = jnp.sum(u * t, axis=-1, keepdims=True)
    return tc_body


def kernel(user_ids, item_ids, user_encodings, item_encodings,
           user_embed, item_embed, user_fc_w, user_fc_b,
           item_fc_w, item_fc_b):
    B = user_ids.shape[0]
    UD = user_encodings.shape[1]
    ID = item_encodings.shape[1]
    ED = user_embed.shape[1]
    PW = 2 * ED
    HID = user_fc_w.shape[1]

    uids = user_ids.astype(jnp.int32)
    iids = item_ids.astype(jnp.int32)
    uq = lax.shift_right_logical(uids, 1)
    iq = lax.shift_right_logical(iids, 1)
    uembp = user_embed.reshape(user_embed.shape[0] // 2, PW)
    iembp = item_embed.reshape(item_embed.shape[0] // 2, PW)

    ue, uemb40, ie, iemb40 = _build_gather(B, UD, ID, PW)(
        uids, iids, uq, iq, user_encodings, item_encodings, uembp, iembp)

    wut, wub = user_fc_w[:UD], user_fc_w[UD:]
    wit, wib = item_fc_w[:ID], item_fc_w[ID:]
    bu = user_fc_b.reshape(1, HID)
    bi = item_fc_b.reshape(1, HID)
    usel = (uids & 1).reshape(B, 1)
    isel = (iids & 1).reshape(B, 1)

    BLK = 2048
    grid = (B // BLK,)
    full = lambda i: (0, 0)
    row = lambda i: (i, 0)
    users, items, ratings = pl.pallas_call(
        _make_tc_body(ED),
        grid=grid,
        in_specs=[
            pl.BlockSpec((BLK, UD), row),
            pl.BlockSpec((BLK, PW), row),
            pl.BlockSpec((BLK, ID), row),
            pl.BlockSpec((BLK, PW), row),
            pl.BlockSpec((BLK, 1), row),
            pl.BlockSpec((BLK, 1), row),
            pl.BlockSpec((UD, HID), full),
            pl.BlockSpec((ED, HID), full),
            pl.BlockSpec((1, HID), full),
            pl.BlockSpec((ID, HID), full),
            pl.BlockSpec((ED, HID), full),
            pl.BlockSpec((1, HID), full),
        ],
        out_specs=[
            pl.BlockSpec((BLK, HID), row),
            pl.BlockSpec((BLK, HID), row),
            pl.BlockSpec((BLK, 1), row),
        ],
        out_shape=(
            jax.ShapeDtypeStruct((B, HID), jnp.float32),
            jax.ShapeDtypeStruct((B, HID), jnp.float32),
            jax.ShapeDtypeStruct((B, 1), jnp.float32),
        ),
    )(ue, uemb40, ie, iemb40, usel, isel, wut, wub, bu, wit, wib, bi)

    return users, items, ratings.reshape(B)


# speedup vs baseline: 1.8552x; 1.8552x over previous
import functools

import jax
import jax.numpy as jnp
from jax import lax
from jax.experimental import pallas as pl
from jax.experimental.pallas import tpu as pltpu
from jax.experimental.pallas import tpu_sc as plsc

_NC = 2
_NS = 16
_NW = _NC * _NS
_CH = 32
_VEC = 16


@functools.lru_cache(maxsize=None)
def _build_gather(B, UD, ID, ED):
    bw = B // _NW
    nchunk = bw // _CH
    mesh = plsc.VectorSubcoreMesh(core_axis_name="c", subcore_axis_name="s")

    @functools.partial(
        pl.kernel,
        out_type=(
            jax.ShapeDtypeStruct((B, UD), jnp.float32),
            jax.ShapeDtypeStruct((B, ED), jnp.float32),
            jax.ShapeDtypeStruct((B, ID), jnp.float32),
            jax.ShapeDtypeStruct((B, ED), jnp.float32),
        ),
        mesh=mesh,
        scratch_types=[
            pltpu.VMEM((bw,), jnp.int32),
            pltpu.VMEM((bw,), jnp.int32),
            pltpu.VMEM((_CH, UD), jnp.float32),
            pltpu.VMEM((_CH, ED), jnp.float32),
            pltpu.VMEM((_CH, ID), jnp.float32),
            pltpu.VMEM((_CH, ED), jnp.float32),
            pltpu.SemaphoreType.DMA,
        ],
    )
    def gather_k(uids, iids, uenc, ienc, uembt, iembt,
                 ue_o, uemb_o, ie_o, iemb_o,
                 uidx, iidx, ue_v, uemb_v, ie_v, iemb_v, sem):
        wid = lax.axis_index("s") * _NC + lax.axis_index("c")
        base = wid * bw
        pltpu.sync_copy(uids.at[pl.ds(base, bw)], uidx)
        pltpu.sync_copy(iids.at[pl.ds(base, bw)], iidx)

        def chunk(c, carry):
            cb = c * _CH
            cps = []
            for v0 in range(0, _CH, _VEC):
                uv = uidx[pl.ds(cb + v0, _VEC)]
                iv = iidx[pl.ds(cb + v0, _VEC)]
                for g in range(_VEC):
                    ru = uv[g]
                    ri = iv[g]
                    d = v0 + g
                    cps.append(pltpu.async_copy(
                        uenc.at[pl.ds(ru, 1)], ue_v.at[pl.ds(d, 1)], sem))
                    cps.append(pltpu.async_copy(
                        uembt.at[pl.ds(ru, 1)], uemb_v.at[pl.ds(d, 1)], sem))
                    cps.append(pltpu.async_copy(
                        ienc.at[pl.ds(ri, 1)], ie_v.at[pl.ds(d, 1)], sem))
                    cps.append(pltpu.async_copy(
                        iembt.at[pl.ds(ri, 1)], iemb_v.at[pl.ds(d, 1)], sem))
            for cp in cps:
                cp.wait()
            s = pl.ds(base + cb, _CH)
            pltpu.sync_copy(ue_v, ue_o.at[s])
            pltpu.sync_copy(uemb_v, uemb_o.at[s])
            pltpu.sync_copy(ie_v, ie_o.at[s])
            pltpu.sync_copy(iemb_v, iemb_o.at[s])
            return carry

        lax.fori_loop(0, nchunk, chunk, 0)

    return gather_k


def _tc_body(ue, uemb, ie, iemb, wut, wub, bu, wit, wib, bi,
             users_o, items_o, ratings_o):
    u = jnp.dot(ue[...], wut[...], preferred_element_type=jnp.float32)
    u = u + jnp.dot(uemb[...], wub[...], preferred_element_type=jnp.float32)
    u = u + bu[...]
    t = jnp.dot(ie[...], wit[...], preferred_element_type=jnp.float32)
    t = t + jnp.dot(iemb[...], wib[...], preferred_element_type=jnp.float32)
    t = t + bi[...]
    users_o[...] = u
    items_o[...] = t
    ratings_o[...] = jnp.sum(u * t, axis=-1, keepdims=True)


def kernel(user_ids, item_ids, user_encodings, item_encodings,
           user_embed, item_embed, user_fc_w, user_fc_b,
           item_fc_w, item_fc_b):
    B = user_ids.shape[0]
    UD = user_encodings.shape[1]
    ID = item_encodings.shape[1]
    ED = user_embed.shape[1]
    HID = user_fc_w.shape[1]

    uids = user_ids.astype(jnp.int32)
    iids = item_ids.astype(jnp.int32)

    ue, uemb, ie, iemb = _build_gather(B, UD, ID, ED)(
        uids, iids, user_encodings, item_encodings, user_embed, item_embed)

    wut, wub = user_fc_w[:UD], user_fc_w[UD:]
    wit, wib = item_fc_w[:ID], item_fc_w[ID:]
    bu = user_fc_b.reshape(1, HID)
    bi = item_fc_b.reshape(1, HID)

    BLK = 2048
    grid = (B // BLK,)
    full = lambda i: (0, 0)
    row = lambda i: (i, 0)
    users, items, ratings = pl.pallas_call(
        _tc_body,
        grid=grid,
        in_specs=[
            pl.BlockSpec((BLK, UD), row),
            pl.BlockSpec((BLK, ED), row),
            pl.BlockSpec((BLK, ID), row),
            pl.BlockSpec((BLK, ED), row),
            pl.BlockSpec((UD, HID), full),
            pl.BlockSpec((ED, HID), full),
            pl.BlockSpec((1, HID), full),
            pl.BlockSpec((ID, HID), full),
            pl.BlockSpec((ED, HID), full),
            pl.BlockSpec((1, HID), full),
        ],
        out_specs=[
            pl.BlockSpec((BLK, HID), row),
            pl.BlockSpec((BLK, HID), row),
            pl.BlockSpec((BLK, 1), row),
        ],
        out_shape=(
            jax.ShapeDtypeStruct((B, HID), jnp.float32),
            jax.ShapeDtypeStruct((B, HID), jnp.float32),
            jax.ShapeDtypeStruct((B, 1), jnp.float32),
        ),
    )(ue, uemb, ie, iemb, wut, wub, bu, wit, wib, bi)

    return users, items, ratings.reshape(B)


# speedup vs baseline: 1.9309x; 1.0408x over previous
import functools

import jax
import jax.numpy as jnp
from jax import lax
from jax.experimental import pallas as pl
from jax.experimental.pallas import tpu as pltpu
from jax.experimental.pallas import tpu_sc as plsc

_NC = 2
_NS = 16
_NW = _NC * _NS
_CH = 32
_VEC = 16


@functools.lru_cache(maxsize=None)
def _build_gather(B, D1, D2):
    bw = B // _NW
    nchunk = bw // _CH
    mesh = plsc.VectorSubcoreMesh(core_axis_name="c", subcore_axis_name="s")

    @functools.partial(
        pl.kernel,
        out_type=(
            jax.ShapeDtypeStruct((B, D1), jnp.float32),
            jax.ShapeDtypeStruct((B, D2), jnp.float32),
        ),
        mesh=mesh,
        scratch_types=[
            pltpu.VMEM((bw,), jnp.int32),
            pltpu.VMEM((_CH, D1), jnp.float32),
            pltpu.VMEM((_CH, D2), jnp.float32),
            pltpu.SemaphoreType.DMA,
        ],
    )
    def gather_k(ids, tab1, tab2, o1, o2, idx, v1, v2, sem):
        wid = lax.axis_index("s") * _NC + lax.axis_index("c")
        base = wid * bw
        pltpu.sync_copy(ids.at[pl.ds(base, bw)], idx)

        def chunk(c, carry):
            cb = c * _CH
            cps = []
            for v0 in range(0, _CH, _VEC):
                v = idx[pl.ds(cb + v0, _VEC)]
                for g in range(_VEC):
                    r = v[g]
                    d = v0 + g
                    cps.append(pltpu.async_copy(
                        tab1.at[pl.ds(r, 1)], v1.at[pl.ds(d, 1)], sem))
                    cps.append(pltpu.async_copy(
                        tab2.at[pl.ds(r, 1)], v2.at[pl.ds(d, 1)], sem))
            for cp in cps:
                cp.wait()
            s = pl.ds(base + cb, _CH)
            pltpu.sync_copy(v1, o1.at[s])
            pltpu.sync_copy(v2, o2.at[s])
            return carry

        lax.fori_loop(0, nchunk, chunk, 0)

    return gather_k


def _tc_body(ue, uemb, ie, iemb, wut, wub, bu, wit, wib, bi,
             usersT_o, itemsT_o, ratingsT_o):
    u = jnp.dot(ue[...], wut[...], preferred_element_type=jnp.float32)
    u = u + jnp.dot(uemb[...], wub[...], preferred_element_type=jnp.float32)
    u = u + bu[...]
    t = jnp.dot(ie[...], wit[...], preferred_element_type=jnp.float32)
    t = t + jnp.dot(iemb[...], wib[...], preferred_element_type=jnp.float32)
    t = t + bi[...]
    usersT_o[...] = u.T
    itemsT_o[...] = t.T
    ratingsT_o[...] = jnp.sum(u * t, axis=-1, keepdims=True).T


def kernel(user_ids, item_ids, user_encodings, item_encodings,
           user_embed, item_embed, user_fc_w, user_fc_b,
           item_fc_w, item_fc_b):
    B = user_ids.shape[0]
    UD = user_encodings.shape[1]
    ID = item_encodings.shape[1]
    ED = user_embed.shape[1]
    HID = user_fc_w.shape[1]

    uids = user_ids.astype(jnp.int32)
    iids = item_ids.astype(jnp.int32)

    ie, iemb = _build_gather(B, ID, ED)(iids, item_encodings, item_embed)
    ue, uemb = _build_gather(B, UD, ED)(uids, user_encodings, user_embed)

    wut, wub = user_fc_w[:UD], user_fc_w[UD:]
    wit, wib = item_fc_w[:ID], item_fc_w[ID:]
    bu = user_fc_b.reshape(1, HID)
    bi = item_fc_b.reshape(1, HID)

    BLK = 2048
    grid = (B // BLK,)
    full = lambda i: (0, 0)
    row = lambda i: (i, 0)
    col = lambda i: (0, i)
    usersT, itemsT, ratingsT = pl.pallas_call(
        _tc_body,
        grid=grid,
        in_specs=[
            pl.BlockSpec((BLK, UD), row),
            pl.BlockSpec((BLK, ED), row),
            pl.BlockSpec((BLK, ID), row),
            pl.BlockSpec((BLK, ED), row),
            pl.BlockSpec((UD, HID), full),
            pl.BlockSpec((ED, HID), full),
            pl.BlockSpec((1, HID), full),
            pl.BlockSpec((ID, HID), full),
            pl.BlockSpec((ED, HID), full),
            pl.BlockSpec((1, HID), full),
        ],
        out_specs=[
            pl.BlockSpec((HID, BLK), col),
            pl.BlockSpec((HID, BLK), col),
            pl.BlockSpec((1, BLK), col),
        ],
        out_shape=(
            jax.ShapeDtypeStruct((HID, B), jnp.float32),
            jax.ShapeDtypeStruct((HID, B), jnp.float32),
            jax.ShapeDtypeStruct((1, B), jnp.float32),
        ),
    )(ue, uemb, ie, iemb, wut, wub, bu, wit, wib, bi)

    return usersT.T, itemsT.T, ratingsT.reshape(B)
